# Initial kernel scaffold; baseline (speedup 1.0000x reference)
import jax
import jax.numpy as jnp
from jax.experimental import pallas as pl


def kernel(x, edge_index, edge_attr, W_e0, b_e0, W_pre0, b_pre0, W_post0, b_post0, W_lin0, b_lin0, gamma0, beta0, W_e1, b_e1, W_pre1, b_pre1, W_post1, b_post1, W_lin1, b_lin1, gamma1, beta1):
    raise NotImplementedError("write your pallas kernel here")


# speedup vs baseline: 1.7868x; 1.7868x over previous
import functools
import math

import jax
import jax.numpy as jnp
from jax.experimental import pallas as pl
from jax.experimental.pallas import tpu as pltpu

_AVG_LOG = math.log(17.0)


def _mm_kernel(x_ref, w_ref, b_ref, o_ref):
    o_ref[...] = (
        jnp.dot(x_ref[...], w_ref[...], preferred_element_type=jnp.float32)
        + b_ref[...]
    )


def _matmul_bias(x, w, b, blk_rows):
    n = x.shape[0]
    f_out = w.shape[1]
    grid = (n // blk_rows,)
    return pl.pallas_call(
        _mm_kernel,
        grid=grid,
        in_specs=[
            pl.BlockSpec((blk_rows, x.shape[1]), lambda i: (i, 0)),
            pl.BlockSpec((w.shape[0], f_out), lambda i: (0, 0)),
            pl.BlockSpec((1, f_out), lambda i: (0, 0)),
        ],
        out_specs=pl.BlockSpec((blk_rows, f_out), lambda i: (i, 0)),
        out_shape=jax.ShapeDtypeStruct((n, f_out), jnp.float32),
    )(x, w, b)


def _edge_kernel(kb, f, dst_ref, src_ref, ea_ref, a_ref, b_ref, wc_ref,
                 sum_ref, msq_ref, mn_ref, mx_ref, cnt_ref, cs_ref):
    @pl.when(pl.program_id(0) == 0)
    def _init():
        sum_ref[...] = jnp.zeros_like(sum_ref)
        msq_ref[...] = jnp.zeros_like(msq_ref)
        mn_ref[...] = jnp.full_like(mn_ref, jnp.inf)
        mx_ref[...] = jnp.full_like(mx_ref, -jnp.inf)

        def zbody(j, c):
            cnt_ref[0, j] = 0
            return c

        jax.lax.fori_loop(0, cnt_ref.shape[1], zbody, 0, unroll=False)

    cs_ref[...] = jnp.dot(ea_ref[...], wc_ref[...],
                          preferred_element_type=jnp.float32)

    def body(i, carry):
        d = dst_ref[0, 0, i]
        s = src_ref[0, 0, i]
        h = (a_ref[pl.ds(d, 1), :] + b_ref[pl.ds(s, 1), :]
             + cs_ref[pl.ds(i, 1), :])
        sum_ref[pl.ds(d, 1), :] += h
        msq_ref[pl.ds(d, 1), :] += h * h
        mn_ref[pl.ds(d, 1), :] = jnp.minimum(mn_ref[pl.ds(d, 1), :], h)
        mx_ref[pl.ds(d, 1), :] = jnp.maximum(mx_ref[pl.ds(d, 1), :], h)
        cnt_ref[0, d] += 1
        return carry

    jax.lax.fori_loop(0, kb, body, 0, unroll=False)


def _edge_aggregate(dst3, src3, edge_attr, a, b, wc, kb):
    n, f = a.shape
    nb = dst3.shape[0]
    kern = functools.partial(_edge_kernel, kb, f)
    full = lambda i: (0, 0)
    out_shapes = [
        jax.ShapeDtypeStruct((n, f), jnp.float32),
        jax.ShapeDtypeStruct((n, f), jnp.float32),
        jax.ShapeDtypeStruct((n, f), jnp.float32),
        jax.ShapeDtypeStruct((n, f), jnp.float32),
        jax.ShapeDtypeStruct((1, n), jnp.int32),
    ]
    return pl.pallas_call(
        kern,
        grid=(nb,),
        in_specs=[
            pl.BlockSpec((1, 1, kb), lambda i: (i, 0, 0),
                         memory_space=pltpu.SMEM),
            pl.BlockSpec((1, 1, kb), lambda i: (i, 0, 0),
                         memory_space=pltpu.SMEM),
            pl.BlockSpec((kb, edge_attr.shape[1]), lambda i: (i, 0)),
            pl.BlockSpec((n, f), full),
            pl.BlockSpec((n, f), full),
            pl.BlockSpec((edge_attr.shape[1], f), full),
        ],
        out_specs=[
            pl.BlockSpec((n, f), full),
            pl.BlockSpec((n, f), full),
            pl.BlockSpec((n, f), full),
            pl.BlockSpec((n, f), full),
            pl.BlockSpec((1, n), full, memory_space=pltpu.SMEM),
        ],
        out_shape=out_shapes,
        scratch_shapes=[pltpu.VMEM((kb, f), jnp.float32)],
        compiler_params=pltpu.CompilerParams(
            vmem_limit_bytes=66_000_000,
        ),
    )(dst3, src3, edge_attr, a, b, wc)


def _post_kernel(avg_log, x_ref, s_ref, q_ref, mn_ref, mx_ref, c_ref,
                 p0_ref, p1_ref, p2_ref, p3_ref, wl_ref, bp_ref, bl_ref,
                 o_ref):
    cnt = c_ref[...]
    deg = jnp.maximum(cnt, 1.0)
    s = s_ref[...]
    mean = s / deg
    msq = q_ref[...] / deg
    var = msq - mean * mean
    std = jnp.sqrt(jnp.maximum(var, 0.0) + 1e-5)
    has = cnt > 0.0
    mn = jnp.where(has, mn_ref[...], 0.0)
    mx = jnp.where(has, mx_ref[...], 0.0)
    agg = jnp.concatenate([mean, mn, mx, std, s], axis=1)
    sl = jnp.log(deg + 1.0)
    t = (
        jnp.dot(x_ref[...], p0_ref[...], preferred_element_type=jnp.float32)
        + jnp.dot(agg, p1_ref[...], preferred_element_type=jnp.float32)
        + (sl / avg_log)
        * jnp.dot(agg, p2_ref[...], preferred_element_type=jnp.float32)
        + (avg_log / sl)
        * jnp.dot(agg, p3_ref[...], preferred_element_type=jnp.float32)
        + bp_ref[...]
    )
    o_ref[...] = (
        jnp.dot(t, wl_ref[...], preferred_element_type=jnp.float32)
        + bl_ref[...]
    )


def _post(x, tabs, p0, p1, p2, p3, wl, bp, bl, blk_rows):
    n, f = x.shape
    s, q, mn, mx, c = tabs
    c = c.reshape(n, 1).astype(jnp.float32)
    full = lambda i: (0, 0)
    row = lambda i: (i, 0)
    kern = functools.partial(_post_kernel, _AVG_LOG)
    return pl.pallas_call(
        kern,
        grid=(n // blk_rows,),
        in_specs=[
            pl.BlockSpec((blk_rows, f), row),
            pl.BlockSpec((blk_rows, f), row),
            pl.BlockSpec((blk_rows, f), row),
            pl.BlockSpec((blk_rows, f), row),
            pl.BlockSpec((blk_rows, f), row),
            pl.BlockSpec((blk_rows, 1), row),
            pl.BlockSpec(p0.shape, full),
            pl.BlockSpec(p1.shape, full),
            pl.BlockSpec(p2.shape, full),
            pl.BlockSpec(p3.shape, full),
            pl.BlockSpec(wl.shape, full),
            pl.BlockSpec((1, f), full),
            pl.BlockSpec((1, f), full),
        ],
        out_specs=pl.BlockSpec((blk_rows, f), row),
        out_shape=jax.ShapeDtypeStruct((n, f), jnp.float32),
    )(x, s, q, mn, mx, c, p0, p1, p2, p3, wl, bp, bl)


def _bn_kernel(t_ref, g_ref, b_ref, o_ref):
    t = t_ref[...]
    m = jnp.mean(t, axis=0, keepdims=True)
    v = jnp.mean((t - m) * (t - m), axis=0, keepdims=True)
    y = (t - m) / jnp.sqrt(v + 1e-5) * g_ref[...] + b_ref[...]
    o_ref[...] = jnp.maximum(y, 0.0)


def _bn_relu(t, gamma, beta):
    n, f = t.shape
    full = lambda: (0, 0)
    return pl.pallas_call(
        _bn_kernel,
        in_specs=[
            pl.BlockSpec((n, f), full),
            pl.BlockSpec((1, f), full),
            pl.BlockSpec((1, f), full),
        ],
        out_specs=pl.BlockSpec((n, f), full),
        out_shape=jax.ShapeDtypeStruct((n, f), jnp.float32),
    )(t, gamma.reshape(1, f), beta.reshape(1, f))


def _pick_block(n, target):
    for cand in range(min(n, target), 0, -1):
        if n % cand == 0 and (cand % 8 == 0 or cand == n):
            return cand
    return n


def _layer(x, dst3, src3, edge_attr, kb, W_e, b_e, W_pre, b_pre, W_post,
           b_post, W_lin, b_lin):
    n, f = x.shape
    w1 = W_pre[:f]
    w2 = W_pre[f:2 * f]
    w3 = W_pre[2 * f:]
    wc = W_e @ w3
    bias_h = (b_pre + b_e @ w3).reshape(1, f)
    wcat = jnp.concatenate([w1, w2], axis=1)
    bcat = jnp.concatenate([bias_h, jnp.zeros_like(bias_h)], axis=1)

    blk = _pick_block(n, 2000)
    ab = _matmul_bias(x, wcat, bcat, blk)
    a = ab[:, :f]
    b = ab[:, f:]

    tabs = _edge_aggregate(dst3, src3, edge_attr, a, b, wc, kb)

    p0 = W_post[:f]
    p1 = W_post[f:f + 5 * f]
    p2 = W_post[f + 5 * f:f + 10 * f]
    p3 = W_post[f + 10 * f:]
    bp = b_post.reshape(1, f)
    bl = b_lin.reshape(1, f)
    return _post(x, tabs, p0, p1, p2, p3, W_lin, bp, bl, blk)


def kernel(x, edge_index, edge_attr, W_e0, b_e0, W_pre0, b_pre0, W_post0,
           b_post0, W_lin0, b_lin0, gamma0, beta0, W_e1, b_e1, W_pre1,
           b_pre1, W_post1, b_post1, W_lin1, b_lin1, gamma1, beta1):
    e = edge_index.shape[1]
    kb = _pick_block(e, 2000)
    nb = e // kb
    src3 = edge_index[0].reshape(nb, 1, kb)
    dst3 = edge_index[1].reshape(nb, 1, kb)

    h = _layer(x, dst3, src3, edge_attr, kb, W_e0, b_e0, W_pre0, b_pre0,
               W_post0, b_post0, W_lin0, b_lin0)
    h = _bn_relu(h, gamma0, beta0)
    h = _layer(h, dst3, src3, edge_attr, kb, W_e1, b_e1, W_pre1, b_pre1,
               W_post1, b_post1, W_lin1, b_lin1)
    h = _bn_relu(h, gamma1, beta1)
    return h
